# baseline (speedup 1.0000x reference)
import functools

import jax
import jax.numpy as jnp
from jax import lax
from jax.experimental import pallas as pl
from jax.experimental.pallas import tpu as pltpu
from jax.experimental.pallas import tpu_sc as plsc

_N_CAT = 26
_N_NUM = 13
_VOCAB = 100000
_D = 64
_B = 4096
_N_TOK = 1 + _N_CAT + _N_NUM

_NC = 2
_NS = 16
_NW = _NC * _NS
_DPW = _D // _NW
_NBV = _B // 16

_VC = 24576
_CHUNKS = [(c * _VC, min(_VC, _VOCAB - c * _VC))
           for c in range((_VOCAB + _VC - 1) // _VC)]

_mesh = plsc.VectorSubcoreMesh(core_axis_name="c", subcore_axis_name="s")


@functools.partial(
    pl.kernel,
    mesh=_mesh,
    compiler_params=pltpu.CompilerParams(
        use_tc_tiling_on_sc=True, needs_layout_passes=False),
    out_type=jax.ShapeDtypeStruct((_N_TOK, _D, _B), jnp.float32),
    scratch_types=[
        pltpu.VMEM((2, _VC), jnp.float32),
        pltpu.VMEM((2, _VC), jnp.float32),
        pltpu.VMEM((2, _VOCAB - 4 * _VC), jnp.float32),
        pltpu.VMEM((2, _B), jnp.float32),
        pltpu.VMEM((2, _B), jnp.float32),
        pltpu.VMEM((_B,), jnp.int32),
        pltpu.VMEM((_B,), jnp.float32),
        pltpu.VMEM((_N_NUM * _D,), jnp.float32),
        pltpu.VMEM((_N_NUM * _D,), jnp.float32),
        pltpu.VMEM((_D,), jnp.float32),
        pltpu.SemaphoreType.DMA,
        pltpu.SemaphoreType.DMA,
        pltpu.SemaphoreType.DMA,
        pltpu.SemaphoreType.DMA,
    ],
)
def _tokenize(cat_hbm, num_hbm, emb_hbm, w_hbm, b_hbm, cls_hbm, out_hbm,
              cb0, cb1, cbt, op0, op1, cat_v, num_v, w_v, b_v, cls_v,
              gs0, gs1, ws0, ws1):
    wid = lax.axis_index("s") * _NC + lax.axis_index("c")
    d0 = wid * _DPW

    pltpu.sync_copy(w_hbm, w_v)
    pltpu.sync_copy(b_hbm, b_v)
    pltpu.sync_copy(cls_hbm, cls_v)

    cbs = (cb0, cb1)
    gsems = (gs0, gs1)
    opairs = (op0, op1)
    wsems = (ws0, ws1)
    write_futs = [None, None]
    oslot = [0]
    iota16 = lax.iota(jnp.int32, 16)

    def acquire_opair():
        sel = oslot[0] % 2
        oslot[0] += 1
        if write_futs[sel] is not None:
            write_futs[sel].wait()
        return sel

    def emit_opair(sel, tok):
        write_futs[sel] = pltpu.async_copy(
            opairs[sel], out_hbm.at[tok, pl.ds(d0, _DPW), :], wsems[sel])

    sel = acquire_opair()
    for dd in range(_DPW):
        csplat = plsc.load_gather(cls_v, [jnp.broadcast_to(d0 + dd, (16,))])

        def body(k, carry, op=opairs[sel], dd=dd, csplat=csplat):
            op[dd, pl.ds(k * 16, 16)] = csplat
            return carry
        lax.fori_loop(0, _NBV, body, 0)
    emit_opair(sel, 0)

    def fire(i, c, slot):
        v0, vlen = _CHUNKS[c]
        dst = cbs[slot] if vlen == _VC else cbt
        return pltpu.async_copy(
            emb_hbm.at[i, pl.ds(d0, _DPW), pl.ds(v0, vlen)],
            dst, gsems[slot])

    for i in range(_N_CAT):
        pltpu.sync_copy(cat_hbm.at[pl.ds(i * _B, _B)], cat_v)
        sel = acquire_opair()
        op = opairs[sel]
        futs = [fire(i, 0, 0), fire(i, 1, 1)]
        for c in range(len(_CHUNKS)):
            slot = c % 2
            v0, vlen = _CHUNKS[c]
            futs[slot].wait()
            cb = cbs[slot] if vlen == _VC else cbt

            def scan(k, carry, cb=cb, op=op, v0=v0, vlen=vlen):
                vcat = cat_v[pl.ds(k * 16, 16)]
                m = (vcat >= v0) & (vcat < v0 + vlen)
                rel = vcat - v0
                pos = iota16 + k * 16
                for dd in range(_DPW):
                    val = plsc.load_gather(
                        cb, [jnp.broadcast_to(dd, (16,)), rel], mask=m)
                    plsc.store_scatter(
                        op, [jnp.broadcast_to(dd, (16,)), pos], val, mask=m)
                return carry
            lax.fori_loop(0, _NBV, scan, 0)
            if c + 2 < len(_CHUNKS):
                futs[slot] = fire(i, c + 2, slot)
        emit_opair(sel, 1 + i)

    for j in range(_N_NUM):
        pltpu.sync_copy(num_hbm.at[pl.ds(j * _B, _B)], num_v)
        sel = acquire_opair()
        op = opairs[sel]
        for dd in range(_DPW):
            jd = jnp.broadcast_to(j * _D + d0 + dd, (16,))
            ws = plsc.load_gather(w_v, [jd])
            bs = plsc.load_gather(b_v, [jd])

            def body(k, carry, op=op, dd=dd, ws=ws, bs=bs):
                nv = num_v[pl.ds(k * 16, 16)]
                op[dd, pl.ds(k * 16, 16)] = nv * ws + bs
                return carry
            lax.fori_loop(0, _NBV, body, 0)
        emit_opair(sel, 1 + _N_CAT + j)

    for sel in range(2):
        if write_futs[sel] is not None:
            write_futs[sel].wait()


def kernel(cat, num, emb_cat, w_num, b_num, cls):
    catT = cat.T.reshape(-1)
    numT = num.T.reshape(-1)
    embT = jnp.transpose(emb_cat, (0, 2, 1))
    outT = _tokenize(catT, numT, embT,
                     w_num.reshape(-1), b_num.reshape(-1), cls.reshape(-1))
    return jnp.transpose(outT, (2, 0, 1))


# speedup vs baseline: 1.9411x; 1.9411x over previous
import functools

import jax
import jax.numpy as jnp
from jax import lax
from jax.experimental import pallas as pl
from jax.experimental.pallas import tpu as pltpu
from jax.experimental.pallas import tpu_sc as plsc

_N_CAT = 26
_N_NUM = 13
_VOCAB = 100000
_D = 64
_B = 4096
_N_TOK = 1 + _N_CAT + _N_NUM

_NC = 2
_NS = 16
_NW = _NC * _NS
_DPW = _D // _NW
_NBV = _B // 16

_mesh = plsc.VectorSubcoreMesh(core_axis_name="c", subcore_axis_name="s")


@functools.partial(
    pl.kernel,
    mesh=_mesh,
    compiler_params=pltpu.CompilerParams(
        use_tc_tiling_on_sc=True, needs_layout_passes=False),
    out_type=jax.ShapeDtypeStruct((_N_TOK, _D, _B), jnp.float32),
    scratch_types=[
        pltpu.VMEM((1, _VOCAB), jnp.float32),
        pltpu.VMEM((_B,), jnp.int32),
        pltpu.VMEM((_B,), jnp.float32),
        pltpu.VMEM((1, _B), jnp.float32),
        pltpu.VMEM((1, _B), jnp.float32),
        pltpu.VMEM((_N_NUM * _D,), jnp.float32),
        pltpu.VMEM((_N_NUM * _D,), jnp.float32),
        pltpu.VMEM((_D,), jnp.float32),
        pltpu.SemaphoreType.DMA,
        pltpu.SemaphoreType.DMA,
    ],
)
def _tokenize(cat_hbm, num_hbm, emb_hbm, w_hbm, b_hbm, cls_hbm, out_hbm,
              col_v, cat_v, num_v, oc0, oc1, w_v, b_v, cls_v, os0, os1):
    wid = lax.axis_index("s") * _NC + lax.axis_index("c")
    d0 = wid * _DPW

    pltpu.sync_copy(w_hbm, w_v)
    pltpu.sync_copy(b_hbm, b_v)
    pltpu.sync_copy(cls_hbm, cls_v)

    ocols = (oc0, oc1)
    osems = (os0, os1)
    write_futs = [None, None]
    slot = [0]

    def emit_column(fill_body, tok, d):
        sel = slot[0] % 2
        slot[0] += 1
        oc = ocols[sel]
        if write_futs[sel] is not None:
            write_futs[sel].wait()
        fill_body(oc)
        write_futs[sel] = pltpu.async_copy(
            oc, out_hbm.at[tok, pl.ds(d, 1), :], osems[sel]
        )

    for dd in range(_DPW):
        d = d0 + dd
        csplat = plsc.load_gather(cls_v, [jnp.broadcast_to(d, (16,))])

        def fill_cls(oc, csplat=csplat):
            def body(k, carry):
                oc[0, pl.ds(k * 16, 16)] = csplat
                return carry
            lax.fori_loop(0, _NBV, body, 0)

        emit_column(fill_cls, 0, d)

    for i in range(_N_CAT):
        pltpu.sync_copy(cat_hbm.at[pl.ds(i * _B, _B)], cat_v)
        for dd in range(_DPW):
            d = d0 + dd
            pltpu.sync_copy(emb_hbm.at[i, pl.ds(d, 1), :], col_v)

            def fill_cat(oc):
                zero16 = jnp.zeros((16,), jnp.int32)

                def body(k, carry):
                    for t in range(4):
                        base = k * 64 + t * 16
                        vcat = cat_v[pl.ds(base, 16)]
                        oc[0, pl.ds(base, 16)] = plsc.load_gather(
                            col_v, [zero16, vcat])
                    return carry
                lax.fori_loop(0, _NBV // 4, body, 0)

            emit_column(fill_cat, 1 + i, d)

    for j in range(_N_NUM):
        pltpu.sync_copy(num_hbm.at[pl.ds(j * _B, _B)], num_v)
        for dd in range(_DPW):
            d = d0 + dd
            jd = jnp.broadcast_to(j * _D + d, (16,))
            ws = plsc.load_gather(w_v, [jd])
            bs = plsc.load_gather(b_v, [jd])

            def fill_num(oc, ws=ws, bs=bs):
                def body(k, carry):
                    nv = num_v[pl.ds(k * 16, 16)]
                    oc[0, pl.ds(k * 16, 16)] = nv * ws + bs
                    return carry
                lax.fori_loop(0, _NBV, body, 0)

            emit_column(fill_num, 1 + _N_CAT + j, d)

    for sel in range(2):
        if write_futs[sel] is not None:
            write_futs[sel].wait()


def kernel(cat, num, emb_cat, w_num, b_num, cls):
    catT = cat.T.reshape(-1)
    numT = num.T.reshape(-1)
    embT = jnp.transpose(emb_cat, (0, 2, 1))
    outT = _tokenize(catT, numT, embT,
                     w_num.reshape(-1), b_num.reshape(-1), cls.reshape(-1))
    return jnp.transpose(outT, (2, 0, 1))


# speedup vs baseline: 2.1695x; 1.1177x over previous
import functools

import jax
import jax.numpy as jnp
from jax import lax
from jax.experimental import pallas as pl
from jax.experimental.pallas import tpu as pltpu
from jax.experimental.pallas import tpu_sc as plsc

_N_CAT = 26
_N_NUM = 13
_VOCAB = 100000
_D = 64
_B = 4096
_N_TOK = 1 + _N_CAT + _N_NUM

_NC = 2
_NS = 16
_NW = _NC * _NS
_DPW = _D // _NW
_NBV = _B // 16

_mesh = plsc.VectorSubcoreMesh(core_axis_name="c", subcore_axis_name="s")


@functools.partial(
    pl.kernel,
    mesh=_mesh,
    compiler_params=pltpu.CompilerParams(
        use_tc_tiling_on_sc=True, needs_layout_passes=False),
    out_type=jax.ShapeDtypeStruct((_N_TOK, _D, _B), jnp.float32),
    scratch_types=[
        pltpu.VMEM((1, _VOCAB), jnp.float32),
        pltpu.VMEM((_B,), jnp.int32),
        pltpu.VMEM((_B,), jnp.int32),
        pltpu.VMEM((_B,), jnp.float32),
        pltpu.VMEM((_B,), jnp.float32),
        pltpu.VMEM((1, _B), jnp.float32),
        pltpu.VMEM((1, _B), jnp.float32),
        pltpu.VMEM((_N_NUM * _D,), jnp.float32),
        pltpu.VMEM((_N_NUM * _D,), jnp.float32),
        pltpu.VMEM((_D,), jnp.float32),
        pltpu.SemaphoreType.DMA,
        pltpu.SemaphoreType.DMA,
        pltpu.SemaphoreType.DMA,
    ],
)
def _tokenize(cat_hbm, num_hbm, emb_hbm, w_hbm, b_hbm, cls_hbm, out_hbm,
              col_v, catA, catB, numA, numB, oc0, oc1, w_v, b_v, cls_v,
              os0, os1, ps):
    wid = lax.axis_index("s") * _NC + lax.axis_index("c")
    d0 = wid * _DPW

    pltpu.sync_copy(w_hbm, w_v)
    pltpu.sync_copy(b_hbm, b_v)
    pltpu.sync_copy(cls_hbm, cls_v)

    ocols = (oc0, oc1)
    osems = (os0, os1)
    write_futs = [None, None]
    slot = [0]

    def emit_column(fill_body, tok, d):
        sel = slot[0] % 2
        slot[0] += 1
        oc = ocols[sel]
        if write_futs[sel] is not None:
            write_futs[sel].wait()
        fill_body(oc)
        write_futs[sel] = pltpu.async_copy(
            oc, out_hbm.at[tok, pl.ds(d, 1), :], osems[sel]
        )

    for dd in range(_DPW):
        d = d0 + dd
        csplat = plsc.load_gather(cls_v, [jnp.broadcast_to(d, (16,))])

        def fill_cls(oc, csplat=csplat):
            def body(k, carry):
                for t in range(4):
                    oc[0, pl.ds(k * 64 + t * 16, 16)] = csplat
                return carry
            lax.fori_loop(0, _NBV // 4, body, 0)

        emit_column(fill_cls, 0, d)

    cats = (catA, catB)
    catfut = pltpu.async_copy(cat_hbm.at[pl.ds(0, _B)], catA, ps)
    for i in range(_N_CAT):
        cat_v = cats[i % 2]
        catfut.wait()
        if i + 1 < _N_CAT:
            catfut = pltpu.async_copy(
                cat_hbm.at[pl.ds((i + 1) * _B, _B)], cats[(i + 1) % 2], ps)
        for dd in range(_DPW):
            d = d0 + dd
            pltpu.sync_copy(emb_hbm.at[i, pl.ds(d, 1), :], col_v)

            def fill_cat(oc, cat_v=cat_v):
                zero16 = jnp.zeros((16,), jnp.int32)

                def body(k, carry):
                    for t in range(4):
                        base = k * 64 + t * 16
                        vcat = cat_v[pl.ds(base, 16)]
                        oc[0, pl.ds(base, 16)] = plsc.load_gather(
                            col_v, [zero16, vcat])
                    return carry
                lax.fori_loop(0, _NBV // 4, body, 0)

            emit_column(fill_cat, 1 + i, d)

    nums = (numA, numB)
    numfut = pltpu.async_copy(num_hbm.at[pl.ds(0, _B)], numA, ps)
    for j in range(_N_NUM):
        num_v = nums[j % 2]
        numfut.wait()
        if j + 1 < _N_NUM:
            numfut = pltpu.async_copy(
                num_hbm.at[pl.ds((j + 1) * _B, _B)], nums[(j + 1) % 2], ps)
        for dd in range(_DPW):
            d = d0 + dd
            jd = jnp.broadcast_to(j * _D + d, (16,))
            ws = plsc.load_gather(w_v, [jd])
            bs = plsc.load_gather(b_v, [jd])

            def fill_num(oc, ws=ws, bs=bs, num_v=num_v):
                def body(k, carry):
                    for t in range(4):
                        base = k * 64 + t * 16
                        nv = num_v[pl.ds(base, 16)]
                        oc[0, pl.ds(base, 16)] = nv * ws + bs
                    return carry
                lax.fori_loop(0, _NBV // 4, body, 0)

            emit_column(fill_num, 1 + _N_CAT + j, d)

    for sel in range(2):
        if write_futs[sel] is not None:
            write_futs[sel].wait()


def kernel(cat, num, emb_cat, w_num, b_num, cls):
    catT = cat.T.reshape(-1)
    numT = num.T.reshape(-1)
    embT = jnp.transpose(emb_cat, (0, 2, 1))
    outT = _tokenize(catT, numT, embT,
                     w_num.reshape(-1), b_num.reshape(-1), cls.reshape(-1))
    return jnp.transpose(outT, (2, 0, 1))
